# baseline (speedup 1.0000x reference)
import jax
import jax.numpy as jnp
from jax import lax
from jax.experimental import pallas as pl
from jax.experimental.pallas import tpu as pltpu
from jax.experimental.pallas import tpu_sc as plsc

VOCAB = 100000
EMBED = 64
BATCH = 4096
SEQLEN = 200
LN_EPS = 1e-5

NC, NS = 2, 16
NW = NC * NS
BW = BATCH // NW
NBG = BW // 16
CP = 1
NCHUNK = SEQLEN // CP
E8 = EMBED // 8


def _body(xT_hbm, tok_hbm, pos_hbm, g_hbm, b_hbm, out_hbm,
          idx_a, idx_b, rows_a, rows_b, res_a, res_b, pos_v, g_v, b_v,
          gsem_a, gsem_b, wsem_a, wsem_b):
    idx_v = (idx_a, idx_b)
    rows_v = (rows_a, rows_b)
    res_v = (res_a, res_b)
    gsem = (gsem_a, gsem_b)
    wsem = (wsem_a, wsem_b)

    cid = lax.axis_index("c")
    sid = lax.axis_index("s")
    wid = sid * NC + cid
    b0 = wid * BW

    pltpu.sync_copy(pos_hbm, pos_v)
    pltpu.sync_copy(g_hbm, g_v)
    pltpu.sync_copy(b_hbm, b_v)
    lanes = lax.iota(jnp.int32, 16)

    def gather_descs(m):
        return [pltpu.make_async_copy(tok_hbm.at[idx_v[m].at[pi]],
                                      rows_v[m].at[pl.ds(pi * BW, BW)],
                                      gsem[m])
                for pi in range(CP)]

    def start_gather(k, m):
        pltpu.sync_copy(
            xT_hbm.at[pl.ds(k * CP, CP), pl.ds(b0, BW)], idx_v[m])
        for d in gather_descs(m):
            d.start()

    def write_desc(k, m):
        return pltpu.make_async_copy(
            res_v[m],
            out_hbm.at[pl.ds(k * CP, CP), pl.ds(0, E8), wid,
                       pl.ds(0, 8), pl.ds(0, 128)],
            wsem[m])

    inv_d = 1.0 / EMBED
    NV = EMBED // 16
    gvec = [g_v[pl.ds(16 * j, 16)] for j in range(NV)]
    bvec = [b_v[pl.ds(16 * j, 16)] for j in range(NV)]

    def compute(k, m):
        rows_x = rows_v[m]
        res_x = res_v[m]
        for pi in range(CP):
            pbase = (k * CP + pi) * EMBED
            pv = [pos_v[pl.ds(pbase + 16 * j, 16)] for j in range(NV)]

            @plsc.parallel_loop(0, BW)
            def _pre(rr):
                r = pi * BW + rr
                for j in range(NV):
                    rows_x[r, pl.ds(16 * j, 16)] = (
                        rows_x[r, pl.ds(16 * j, 16)] + pv[j])

            @plsc.parallel_loop(0, NBG)
            def _bg(bg):
                ri = pi * BW + bg * 16 + lanes
                acc = None
                acc2 = None
                for e in range(EMBED):
                    ce = jnp.full((16,), e, jnp.int32)
                    hv = plsc.load_gather(rows_x, [ri, ce])
                    acc = hv if acc is None else acc + hv
                    sq = hv * hv
                    acc2 = sq if acc2 is None else acc2 + sq
                mean = acc * inv_d
                var = acc2 * inv_d - mean * mean
                xv = var + LN_EPS
                iv = plsc.bitcast(xv, jnp.int32)
                iv = 0x5F3759DF - lax.shift_right_logical(iv, 1)
                y = plsc.bitcast(iv, jnp.float32)
                hx = 0.5 * xv
                y = y * (1.5 - hx * y * y)
                y = y * (1.5 - hx * y * y)
                y = y * (1.5 - hx * y * y)
                for e in range(EMBED):
                    ce = jnp.full((16,), e, jnp.int32)
                    hv = plsc.load_gather(rows_x, [ri, ce])
                    res_x[pi, e // 8, e % 8, pl.ds(bg * 16, 16)] = (
                        (hv - mean) * y * gvec[e // 16][e % 16]
                        + bvec[e // 16][e % 16])

    start_gather(0, 0)

    @pl.loop(0, NCHUNK, step=2)
    def _chunks(c):
        for b in range(2):
            m = b
            k = c + b

            @pl.when(k + 1 < NCHUNK)
            def _():
                start_gather(k + 1, 1 - m)

            for d in gather_descs(m):
                d.wait()

            @pl.when(k >= 2)
            def _():
                write_desc(0, m).wait()
            compute(k, m)
            write_desc(k, m).start()

    write_desc(0, 0).wait()
    write_desc(0, 1).wait()


_sc_call = pl.kernel(
    _body,
    out_type=jax.ShapeDtypeStruct((SEQLEN, E8, NW, 8, 128), jnp.float32),
    mesh=plsc.VectorSubcoreMesh(core_axis_name="c", subcore_axis_name="s"),
    scratch_types=[
        pltpu.VMEM((CP, BW), jnp.int32),
        pltpu.VMEM((CP, BW), jnp.int32),
        pltpu.VMEM((CP * BW, EMBED), jnp.float32),
        pltpu.VMEM((CP * BW, EMBED), jnp.float32),
        pltpu.VMEM((CP, E8, 8, 128), jnp.float32),
        pltpu.VMEM((CP, E8, 8, 128), jnp.float32),
        pltpu.VMEM((SEQLEN * EMBED,), jnp.float32),
        pltpu.VMEM((EMBED,), jnp.float32),
        pltpu.VMEM((EMBED,), jnp.float32),
        pltpu.SemaphoreType.DMA,
        pltpu.SemaphoreType.DMA,
        pltpu.SemaphoreType.DMA,
        pltpu.SemaphoreType.DMA,
    ],
    compiler_params=pltpu.CompilerParams(needs_layout_passes=False,
                                         use_tc_tiling_on_sc=False),
)


def kernel(x, token_table, pos_table, ln_gamma, ln_beta):
    xT = x.T
    pos_flat = pos_table[:SEQLEN].reshape(SEQLEN * EMBED)
    out5 = _sc_call(xT, token_table, pos_flat, ln_gamma, ln_beta)
    out = out5.transpose(2, 4, 0, 1, 3).reshape(BATCH, SEQLEN, EMBED)
    return out, x > 0


# speedup vs baseline: 2.8984x; 2.8984x over previous
import jax
import jax.numpy as jnp
from jax import lax
from jax.experimental import pallas as pl
from jax.experimental.pallas import tpu as pltpu
from jax.experimental.pallas import tpu_sc as plsc

VOCAB = 100000
EMBED = 64
BATCH = 4096
SEQLEN = 200
LN_EPS = 1e-5

NC, NS = 2, 16
NW = NC * NS
SEQ_W = BATCH // NW
CSEQ = 2
CROWS = CSEQ * SEQLEN
CELEMS = CROWS * EMBED
NCHUNK = SEQ_W // CSEQ
NVEC = EMBED // 16
GSPLIT = ((0, 128), (128, SEQLEN - 128))


def _body(x_hbm, tok_hbm, pos_hbm, g_hbm, b_hbm, out_hbm,
          idx_a, idx_b, rows_a, rows_b, res_a, res_b, pos_v, g_v, b_v,
          gsem_a, gsem_b, wsem_a, wsem_b):
    idx_v = (idx_a, idx_b)
    rows_v = (rows_a, rows_b)
    res_v = (res_a, res_b)
    gsem = (gsem_a, gsem_b)
    wsem = (wsem_a, wsem_b)

    cid = lax.axis_index("c")
    sid = lax.axis_index("s")
    wid = sid * NC + cid
    seq0 = wid * SEQ_W

    pltpu.sync_copy(pos_hbm, pos_v)
    pltpu.sync_copy(g_hbm, g_v)
    pltpu.sync_copy(b_hbm, b_v)
    gamma = [g_v[pl.ds(16 * j, 16)] for j in range(NVEC)]
    beta = [b_v[pl.ds(16 * j, 16)] for j in range(NVEC)]

    def gather_descs(m):
        return [pltpu.make_async_copy(
                    tok_hbm.at[idx_v[m].at[pl.ds(s * SEQLEN + off, n)]],
                    rows_v[m].at[pl.ds(s * SEQLEN + off, n)],
                    gsem[m])
                for s in range(CSEQ) for off, n in GSPLIT]

    def start_gather(k, m):
        pltpu.sync_copy(
            x_hbm.at[pl.ds((seq0 + k * CSEQ) * SEQLEN, CROWS)], idx_v[m])
        for d in gather_descs(m):
            d.start()

    def write_desc(k, m):
        return pltpu.make_async_copy(
            res_v[m],
            out_hbm.at[pl.ds((seq0 + k * CSEQ) * SEQLEN * EMBED, CELEMS)],
            wsem[m])

    inv_d = 1.0 / EMBED

    def compute(m):
        rows_x = rows_v[m]
        res_x = res_v[m]

        @plsc.parallel_loop(0, SEQLEN, unroll=2)
        def _row(p):
            pv = [pos_v[pl.ds(p * EMBED + 16 * j, 16)] for j in range(NVEC)]
            for s in range(CSEQ):
                r = s * SEQLEN + p
                h = [rows_x[r, pl.ds(16 * j, 16)] + pv[j]
                     for j in range(NVEC)]
                s1 = jnp.sum((h[0] + h[1]) + (h[2] + h[3]))
                s2 = jnp.sum((h[0] * h[0] + h[1] * h[1])
                             + (h[2] * h[2] + h[3] * h[3]))
                mean = s1 * inv_d
                var = s2 * inv_d - mean * mean
                xv = var + LN_EPS
                i = lax.bitcast_convert_type(xv, jnp.int32)
                i = 0x5F3759DF - lax.shift_right_logical(i, 1)
                y = lax.bitcast_convert_type(i, jnp.float32)
                hx = 0.5 * xv
                y = y * (1.5 - hx * y * y)
                y = y * (1.5 - hx * y * y)
                y = y * (1.5 - hx * y * y)
                for j in range(NVEC):
                    res_x[pl.ds(r * EMBED + 16 * j, 16)] = (
                        ((h[j] - mean) * y) * gamma[j] + beta[j])

    start_gather(0, 0)

    @pl.loop(0, NCHUNK, step=2)
    def _chunks(c):
        for b in range(2):
            m = b
            k = c + b

            @pl.when(k + 1 < NCHUNK)
            def _():
                start_gather(k + 1, 1 - m)

            for d in gather_descs(m):
                d.wait()

            @pl.when(k >= 2)
            def _():
                write_desc(0, m).wait()
            compute(m)
            write_desc(k, m).start()

    write_desc(0, 0).wait()
    write_desc(0, 1).wait()


_sc_call = pl.kernel(
    _body,
    out_type=jax.ShapeDtypeStruct((BATCH * SEQLEN * EMBED,), jnp.float32),
    mesh=plsc.VectorSubcoreMesh(core_axis_name="c", subcore_axis_name="s"),
    scratch_types=[
        pltpu.VMEM((CROWS,), jnp.int32),
        pltpu.VMEM((CROWS,), jnp.int32),
        pltpu.VMEM((CROWS, EMBED), jnp.float32),
        pltpu.VMEM((CROWS, EMBED), jnp.float32),
        pltpu.VMEM((CELEMS,), jnp.float32),
        pltpu.VMEM((CELEMS,), jnp.float32),
        pltpu.VMEM((SEQLEN * EMBED,), jnp.float32),
        pltpu.VMEM((EMBED,), jnp.float32),
        pltpu.VMEM((EMBED,), jnp.float32),
        pltpu.SemaphoreType.DMA,
        pltpu.SemaphoreType.DMA,
        pltpu.SemaphoreType.DMA,
        pltpu.SemaphoreType.DMA,
    ],
    compiler_params=pltpu.CompilerParams(needs_layout_passes=False,
                                         use_tc_tiling_on_sc=False),
)


def kernel(x, token_table, pos_table, ln_gamma, ln_beta):
    x_flat = x.reshape(BATCH * SEQLEN)
    pos_flat = pos_table[:SEQLEN].reshape(SEQLEN * EMBED)
    out = _sc_call(x_flat, token_table, pos_flat, ln_gamma, ln_beta)
    return out.reshape(BATCH, SEQLEN, EMBED), x > 0
